# Initial kernel scaffold; baseline (speedup 1.0000x reference)
import jax
import jax.numpy as jnp
from jax.experimental import pallas as pl


def kernel(user_embedding, item_embedding, edge_index, edge_weight):
    raise NotImplementedError("write your pallas kernel here")


# speedup vs baseline: 5.4158x; 5.4158x over previous
import functools

import jax
import jax.numpy as jnp
from jax import lax
from jax.experimental import pallas as pl
from jax.experimental.pallas import tpu as pltpu
from jax.experimental.pallas import tpu_sc as plsc

N_USERS = 4000
N_ITEMS = 6000
N_NODES = N_USERS + N_ITEMS
N_EDGES = 320000
D = 128
LANES = 16

NC = 2
NS = 16
NW = NC * NS
E_PER_W = N_EDGES // NW
CHUNK = 128
N_FULL = E_PER_W // CHUNK
TAIL = E_PER_W - N_FULL * CHUNK
ROWS_PER_SUB = 624
ROWS_REM = N_NODES - NS * ROWS_PER_SUB

_mesh = plsc.VectorSubcoreMesh(core_axis_name="c", subcore_axis_name="s")


@functools.partial(
    pl.kernel,
    out_type=jax.ShapeDtypeStruct((NC * N_NODES, D), jnp.float32),
    mesh=_mesh,
    scratch_types=[
        pltpu.VMEM((CHUNK,), jnp.int32),
        pltpu.VMEM((CHUNK,), jnp.int32),
        pltpu.VMEM((CHUNK,), jnp.float32),
        pltpu.VMEM((CHUNK, D), jnp.float32),
        pltpu.VMEM((TAIL,), jnp.int32),
        pltpu.VMEM((TAIL,), jnp.int32),
        pltpu.VMEM((TAIL,), jnp.float32),
        pltpu.VMEM((TAIL, D), jnp.float32),
        pltpu.VMEM_SHARED((N_NODES, D), jnp.float32),
        pltpu.SemaphoreType.DMA,
    ],
)
def _sc_scatter(h, src, dst, w, zeros, out, src_v, dst_v, w_v, rows_v,
                src_t, dst_t, w_t, rows_t, acc, sem):
    cid = lax.axis_index("c")
    sid = lax.axis_index("s")
    ebase = (cid * NS + sid) * E_PER_W

    pltpu.sync_copy(zeros, acc.at[pl.ds(sid * ROWS_PER_SUB, ROWS_PER_SUB)])

    @pl.when(sid == NS - 1)
    def _zero_rem():
        pltpu.sync_copy(zeros.at[pl.ds(0, ROWS_REM)],
                        acc.at[pl.ds(NS * ROWS_PER_SUB, ROWS_REM)])

    plsc.subcore_barrier()

    def scale(rows_ref, w_ref, n_edges):
        def group(g, _):
            wv = w_ref[pl.ds(g * LANES, LANES)]
            for e in range(LANES):
                row = g * LANES + e
                ws = jnp.full((LANES,), wv[e], jnp.float32)
                for j in range(D // LANES):
                    sl = pl.ds(j * LANES, LANES)
                    rows_ref[row, sl] = rows_ref[row, sl] * ws
            return 0
        lax.fori_loop(0, n_edges // LANES, group, 0)

    def process(base, src_r, dst_r, w_r, rows_r, n_edges):
        pltpu.sync_copy(src.at[pl.ds(base, n_edges)], src_r)
        pltpu.sync_copy(dst.at[pl.ds(base, n_edges)], dst_r)
        pltpu.sync_copy(w.at[pl.ds(base, n_edges)], w_r)
        pltpu.async_copy(h.at[src_r], rows_r, sem).wait()
        scale(rows_r, w_r, n_edges)
        pltpu.sync_copy(rows_r, acc.at[dst_r], add=True)

    def chunk_body(k, _):
        process(ebase + k * CHUNK, src_v, dst_v, w_v, rows_v, CHUNK)
        return 0

    lax.fori_loop(0, N_FULL, chunk_body, 0)
    process(ebase + N_FULL * CHUNK, src_t, dst_t, w_t, rows_t, TAIL)

    plsc.subcore_barrier()
    rbase = sid * ROWS_PER_SUB
    pltpu.sync_copy(acc.at[pl.ds(rbase, ROWS_PER_SUB)],
                    out.at[pl.ds(cid * N_NODES + rbase, ROWS_PER_SUB)])

    @pl.when(sid == NS - 1)
    def _flush_rem():
        pltpu.sync_copy(acc.at[pl.ds(NS * ROWS_PER_SUB, ROWS_REM)],
                        out.at[pl.ds(cid * N_NODES + NS * ROWS_PER_SUB, ROWS_REM)])


_TC_ROWS = 1000


def _tc_finalize_body(a_ref, b_ref, o_ref):
    s = a_ref[...] + b_ref[...]
    n2 = jnp.sum(s * s, axis=1, keepdims=True)
    o_ref[...] = s / jnp.maximum(jnp.sqrt(n2), 1e-12)


_tc_finalize = pl.pallas_call(
    _tc_finalize_body,
    grid=(N_NODES // _TC_ROWS,),
    in_specs=[
        pl.BlockSpec((_TC_ROWS, D), lambda i: (i, 0)),
        pl.BlockSpec((_TC_ROWS, D), lambda i: (i + N_NODES // _TC_ROWS, 0)),
    ],
    out_specs=pl.BlockSpec((_TC_ROWS, D), lambda i: (i, 0)),
    out_shape=jax.ShapeDtypeStruct((N_NODES, D), jnp.float32),
)


def kernel(user_embedding, item_embedding, edge_index, edge_weight):
    h = jnp.concatenate([user_embedding, item_embedding], axis=0)
    src = edge_index[0].astype(jnp.int32)
    dst = edge_index[1].astype(jnp.int32)
    w = edge_weight.astype(jnp.float32)
    zeros = jnp.zeros((ROWS_PER_SUB, D), jnp.float32)
    partials = _sc_scatter(h, src, dst, w, zeros)
    return _tc_finalize(partials, partials)


# speedup vs baseline: 9.9007x; 1.8281x over previous
import functools

import jax
import jax.numpy as jnp
from jax import lax
from jax.experimental import pallas as pl
from jax.experimental.pallas import tpu as pltpu
from jax.experimental.pallas import tpu_sc as plsc

N_USERS = 4000
N_ITEMS = 6000
N_NODES = N_USERS + N_ITEMS
N_EDGES = 320000
D = 128
LANES = 16

NC = 2
NS = 16
NW = NC * NS
CHUNK = 128
N_CHUNKS = N_EDGES // CHUNK
N_POS = N_CHUNKS // NW
N_EXTRA = N_CHUNKS - N_POS * NW
ROWS_PER_SUB = 624
ROWS_REM = N_NODES - NS * ROWS_PER_SUB

_mesh = plsc.VectorSubcoreMesh(core_axis_name="c", subcore_axis_name="s")


@functools.partial(
    pl.kernel,
    out_type=jax.ShapeDtypeStruct((NC * N_NODES, D), jnp.float32),
    mesh=_mesh,
    scratch_types=[
        pltpu.VMEM((3, CHUNK), jnp.int32),
        pltpu.VMEM((3, CHUNK), jnp.int32),
        pltpu.VMEM((CHUNK, D), jnp.float32),
        pltpu.VMEM((CHUNK, D), jnp.float32),
        pltpu.VMEM_SHARED((N_NODES, D), jnp.float32),
        pltpu.SemaphoreType.DMA,
        pltpu.SemaphoreType.DMA,
    ],
)
def _sc_scatter(h, packed, zeros, out, pk0, pk1, rows0, rows1, acc, sem0, sem1):
    cid = lax.axis_index("c")
    sid = lax.axis_index("s")
    wid = cid * NS + sid
    pks = (pk0, pk1)
    rows = (rows0, rows1)
    sems = (sem0, sem1)

    pltpu.sync_copy(zeros, acc.at[pl.ds(sid * ROWS_PER_SUB, ROWS_PER_SUB)])

    @pl.when(sid == NS - 1)
    def _zero_rem():
        pltpu.sync_copy(zeros.at[pl.ds(0, ROWS_REM)],
                        acc.at[pl.ds(NS * ROWS_PER_SUB, ROWS_REM)])

    plsc.subcore_barrier()

    def scale(rows_ref, pk_ref):
        def group(g, _):
            wv = pk_ref[2, pl.ds(g * LANES, LANES)].view(jnp.float32)
            for e in range(LANES):
                row = g * LANES + e
                ws = jnp.full((LANES,), wv[e], jnp.float32)
                for j in range(D // LANES):
                    sl = pl.ds(j * LANES, LANES)
                    rows_ref[row, sl] = rows_ref[row, sl] * ws
            return 0
        lax.fori_loop(0, CHUNK // LANES, group, 0)

    def load_idx(t, pk_ref):
        base = (wid + NW * t) * CHUNK
        pltpu.sync_copy(packed.at[:, pl.ds(base, CHUNK)], pk_ref)

    def start_gather(b):
        pltpu.async_copy(h.at[pks[b].at[0]], rows[b], sems[b])

    def finish_chunk(b):
        pltpu.make_async_copy(h.at[pks[b].at[0]], rows[b], sems[b]).wait()
        scale(rows[b], pks[b])
        pltpu.sync_copy(rows[b], acc.at[pks[b].at[1]], add=True)

    for b in range(2):
        load_idx(b, pks[b])
        start_gather(b)

    def pair_body(i, _):
        for b in range(2):
            finish_chunk(b)
            load_idx(2 * i + b + 2, pks[b])
            start_gather(b)
        return 0

    lax.fori_loop(0, N_POS // 2 - 1, pair_body, 0)

    for b in range(2):
        finish_chunk(b)

    @pl.when(wid < N_EXTRA)
    def _extra():
        load_idx(N_POS, pk0)
        pltpu.async_copy(h.at[pk0.at[0]], rows0, sem0).wait()
        scale(rows0, pk0)
        pltpu.sync_copy(rows0, acc.at[pk0.at[1]], add=True)

    plsc.subcore_barrier()
    rbase = sid * ROWS_PER_SUB
    pltpu.sync_copy(acc.at[pl.ds(rbase, ROWS_PER_SUB)],
                    out.at[pl.ds(cid * N_NODES + rbase, ROWS_PER_SUB)])

    @pl.when(sid == NS - 1)
    def _flush_rem():
        pltpu.sync_copy(acc.at[pl.ds(NS * ROWS_PER_SUB, ROWS_REM)],
                        out.at[pl.ds(cid * N_NODES + NS * ROWS_PER_SUB, ROWS_REM)])


_TC_ROWS = 1000


def _tc_finalize_body(a_ref, b_ref, o_ref):
    s = a_ref[...] + b_ref[...]
    n2 = jnp.sum(s * s, axis=1, keepdims=True)
    o_ref[...] = s / jnp.maximum(jnp.sqrt(n2), 1e-12)


_tc_finalize = pl.pallas_call(
    _tc_finalize_body,
    grid=(N_NODES // _TC_ROWS,),
    in_specs=[
        pl.BlockSpec((_TC_ROWS, D), lambda i: (i, 0)),
        pl.BlockSpec((_TC_ROWS, D), lambda i: (i + N_NODES // _TC_ROWS, 0)),
    ],
    out_specs=pl.BlockSpec((_TC_ROWS, D), lambda i: (i, 0)),
    out_shape=jax.ShapeDtypeStruct((N_NODES, D), jnp.float32),
)


def kernel(user_embedding, item_embedding, edge_index, edge_weight):
    h = jnp.concatenate([user_embedding, item_embedding], axis=0)
    src = edge_index[0].astype(jnp.int32)
    dst = edge_index[1].astype(jnp.int32)
    wbits = lax.bitcast_convert_type(edge_weight.astype(jnp.float32), jnp.int32)
    packed = jnp.stack([src, dst, wbits])
    zeros = jnp.zeros((ROWS_PER_SUB, D), jnp.float32)
    partials = _sc_scatter(h, packed, zeros)
    return _tc_finalize(partials, partials)
